# baseline (speedup 1.0000x reference)
import functools

import numpy as np
import jax
import jax.numpy as jnp
from jax import lax
from jax.experimental import pallas as pl
from jax.experimental.pallas import tpu as pltpu
from jax.experimental.pallas import tpu_sc as plsc

EMBED_DIM = 128
HIDDEN = 256
NUM_CLASSES = 119
PADC = 128
SEG = 1024
MASK_RATE = 0.3
NSEG = 16
N = NSEG * SEG
PER_SEG = max(int(SEG * MASK_RATE), 1)
TOTAL = NSEG * PER_SEG

TASK_COL = 0
NTILES = 16
L = 16
NVEC = 20


def _np_uniform_threefry(seed: int, n: int) -> np.ndarray:
    def rotl(x, d):
        return ((x << np.uint32(d)) | (x >> np.uint32(32 - d))).astype(np.uint32)

    counts = np.arange(n, dtype=np.uint64)
    x = [(counts >> np.uint64(32)).astype(np.uint32),
         (counts & np.uint64(0xFFFFFFFF)).astype(np.uint32)]
    k0, k1 = np.uint32(0), np.uint32(seed)
    rotations = ((13, 15, 26, 6), (17, 29, 16, 24))
    ks = (k0, k1, np.uint32(k0 ^ k1 ^ 0x1BD11BDA))
    x[0] = (x[0] + ks[0]).astype(np.uint32)
    x[1] = (x[1] + ks[1]).astype(np.uint32)
    for i in range(5):
        for r in rotations[i % 2]:
            x[0] = (x[0] + x[1]).astype(np.uint32)
            x[1] = rotl(x[1], r) ^ x[0]
        x[0] = (x[0] + ks[(i + 1) % 3]).astype(np.uint32)
        x[1] = (x[1] + ks[(i + 2) % 3] + np.uint32(i + 1)).astype(np.uint32)
    bits = x[0] ^ x[1]
    f = ((bits >> np.uint32(9)) | np.uint32(0x3F800000)).view(np.float32) - np.float32(1.0)
    return np.maximum(np.float32(0.0), f)


_U = _np_uniform_threefry(42, TOTAL)
_U_PAD = np.zeros((NTILES, NVEC * L), dtype=np.float32)
_U_PAD[:, :PER_SEG] = _U.reshape(NTILES, PER_SEG)
_U_PAD = _U_PAD.reshape(NTILES, NVEC, L)


def _head_body(bm_ref, w1_ref, b1_ref, w2_ref, b2_ref, s_ref, lz_ref):
    rb = jnp.maximum(bm_ref[...].reshape(1, EMBED_DIM), 0.0)
    hid = jnp.dot(rb, w1_ref[...], preferred_element_type=jnp.float32)
    hid = jnp.maximum(hid + b1_ref[...].reshape(1, HIDDEN), 0.0)
    s = jnp.dot(hid, w2_ref[...], preferred_element_type=jnp.float32)
    s = s + b2_ref[...].reshape(1, NUM_CLASSES)
    m = jnp.max(s)
    lz = m + jnp.log(jnp.sum(jnp.exp(s - m)))
    sp = jnp.concatenate(
        [s, jnp.full((1, PADC - NUM_CLASSES), -1e30, jnp.float32)], axis=1)
    s_ref[...] = jnp.broadcast_to(sp, (8, PADC))
    lz_ref[...] = jnp.full((8, PADC), lz, dtype=jnp.float32)


_head = pl.pallas_call(
    _head_body,
    out_shape=[
        jax.ShapeDtypeStruct((8, PADC), jnp.float32),
        jax.ShapeDtypeStruct((8, PADC), jnp.float32),
    ],
)


_mesh = plsc.VectorSubcoreMesh(core_axis_name="c", subcore_axis_name="s")


@functools.partial(
    pl.kernel,
    mesh=_mesh,
    out_type=jax.ShapeDtypeStruct((L,), jnp.float32),
    scratch_types=[
        pltpu.VMEM((NVEC, L), jnp.float32),
        pltpu.VMEM((24,), jnp.int32),
        pltpu.VMEM((NVEC * L,), jnp.float32),
        pltpu.VMEM((PADC,), jnp.float32),
        pltpu.VMEM((PADC,), jnp.float32),
        pltpu.VMEM((L,), jnp.float32),
        pltpu.VMEM((NTILES, L), jnp.float32),
        pltpu.VMEM_SHARED((NTILES, L), jnp.float32),
        pltpu.SemaphoreType.DMA,
    ],
    compiler_params=pltpu.CompilerParams(needs_layout_passes=False),
)
def _sc_loss(u_hbm, ptr_hbm, xf_hbm, s_hbm, lz_hbm, out_hbm,
             u_v, ptr_v, lbl_v, s_v, lz_v, acc_v, sums_v, shared, sem):
    cid = lax.axis_index("c")
    sid = lax.axis_index("s")

    @pl.when(cid == 0)
    def _core0():
        pltpu.sync_copy(u_hbm.at[sid], u_v)
        pltpu.sync_copy(ptr_hbm, ptr_v.at[pl.ds(0, NSEG + 1)])
        lo = plsc.load_gather(ptr_v, [jnp.full((L,), sid, jnp.int32)])
        hi = plsc.load_gather(ptr_v, [jnp.full((L,), sid + 1, jnp.int32)])
        szf = (hi - lo).astype(jnp.float32)

        copies = []
        for j in range(NVEC):
            elem = (u_v[j] * szf).astype(jnp.int32) + lo
            copies.append(
                pltpu.async_copy(xf_hbm.at[elem], lbl_v.at[pl.ds(j * L, L)], sem))
        pltpu.sync_copy(s_hbm.at[0], s_v)
        for cp in copies:
            cp.wait()

        iota = lax.iota(jnp.int32, L)
        acc = jnp.zeros((L,), jnp.float32)
        for j in range(NVEC):
            li = lbl_v[pl.ds(j * L, L)].astype(jnp.int32)
            sv = plsc.load_gather(s_v, [li])
            acc = acc + jnp.where((j * L) + iota < PER_SEG, sv, 0.0)
        acc_v[...] = acc
        pltpu.sync_copy(acc_v, shared.at[sid])
        plsc.subcore_barrier()

        @pl.when(sid == 0)
        def _reduce():
            pltpu.sync_copy(shared, sums_v)
            pltpu.sync_copy(lz_hbm.at[0], lz_v)
            tot = jnp.zeros((L,), jnp.float32)
            for r in range(NTILES):
                tot = tot + sums_v[r]
            total = jnp.sum(tot)
            lzv = lz_v[pl.ds(0, L)]
            acc_v[...] = lzv - total * (1.0 / TOTAL)
            pltpu.sync_copy(acc_v, out_hbm)


def kernel(x, ptr, W_model, b_model, W1, b1, W2, b2):
    s_arr, lz_arr = _head(b_model, W1, b1, W2, b2)

    u_c = jnp.asarray(_U_PAD)
    out = _sc_loss(u_c, ptr, x[:, TASK_COL], s_arr, lz_arr)
    return out[0]


# speedup vs baseline: 1.1711x; 1.1711x over previous
import functools

import numpy as np
import jax
import jax.numpy as jnp
from jax import lax
from jax.experimental import pallas as pl
from jax.experimental.pallas import tpu as pltpu
from jax.experimental.pallas import tpu_sc as plsc

EMBED_DIM = 128
HIDDEN = 256
NUM_CLASSES = 119
PADC = 128
SEG = 1024
MASK_RATE = 0.3
NSEG = 16
N = NSEG * SEG
PER_SEG = max(int(SEG * MASK_RATE), 1)
TOTAL = NSEG * PER_SEG

TASK_COL = 0
NTILES = 16
L = 16
NVEC = 20


def _np_uniform_threefry(seed: int, n: int) -> np.ndarray:
    def rotl(x, d):
        return ((x << np.uint32(d)) | (x >> np.uint32(32 - d))).astype(np.uint32)

    counts = np.arange(n, dtype=np.uint64)
    x = [(counts >> np.uint64(32)).astype(np.uint32),
         (counts & np.uint64(0xFFFFFFFF)).astype(np.uint32)]
    k0, k1 = np.uint32(0), np.uint32(seed)
    rotations = ((13, 15, 26, 6), (17, 29, 16, 24))
    ks = (k0, k1, np.uint32(k0 ^ k1 ^ 0x1BD11BDA))
    x[0] = (x[0] + ks[0]).astype(np.uint32)
    x[1] = (x[1] + ks[1]).astype(np.uint32)
    for i in range(5):
        for r in rotations[i % 2]:
            x[0] = (x[0] + x[1]).astype(np.uint32)
            x[1] = rotl(x[1], r) ^ x[0]
        x[0] = (x[0] + ks[(i + 1) % 3]).astype(np.uint32)
        x[1] = (x[1] + ks[(i + 2) % 3] + np.uint32(i + 1)).astype(np.uint32)
    bits = x[0] ^ x[1]
    f = ((bits >> np.uint32(9)) | np.uint32(0x3F800000)).view(np.float32) - np.float32(1.0)
    return np.maximum(np.float32(0.0), f)


_U = _np_uniform_threefry(42, TOTAL)
_U_PAD = np.zeros((NTILES, NVEC * L), dtype=np.float32)
_U_PAD[:, :PER_SEG] = _U.reshape(NTILES, PER_SEG)
_U_PAD = _U_PAD.reshape(NTILES, NVEC, L)


def _head_body(bm_ref, w1_ref, b1_ref, w2_ref, b2_ref, s_ref, lz_ref):
    rb = jnp.maximum(bm_ref[...].reshape(1, EMBED_DIM), 0.0)
    hid = jnp.dot(rb, w1_ref[...], preferred_element_type=jnp.float32)
    hid = jnp.maximum(hid + b1_ref[...].reshape(1, HIDDEN), 0.0)
    s = jnp.dot(hid, w2_ref[...], preferred_element_type=jnp.float32)
    s = s + b2_ref[...].reshape(1, NUM_CLASSES)
    m = jnp.max(s)
    lz = m + jnp.log(jnp.sum(jnp.exp(s - m)))
    sp = jnp.concatenate(
        [s, jnp.full((1, PADC - NUM_CLASSES), -1e30, jnp.float32)], axis=1)
    s_ref[...] = jnp.broadcast_to(sp, (8, PADC))
    lz_ref[...] = jnp.full((8, PADC), lz, dtype=jnp.float32)


_head = pl.pallas_call(
    _head_body,
    out_shape=[
        jax.ShapeDtypeStruct((8, PADC), jnp.float32),
        jax.ShapeDtypeStruct((8, PADC), jnp.float32),
    ],
)


_mesh = plsc.VectorSubcoreMesh(core_axis_name="c", subcore_axis_name="s")


@functools.partial(
    pl.kernel,
    mesh=_mesh,
    out_type=jax.ShapeDtypeStruct((L,), jnp.float32),
    scratch_types=[
        pltpu.VMEM((NVEC, L), jnp.float32),
        pltpu.VMEM((24,), jnp.int32),
        pltpu.VMEM((NVEC * L,), jnp.float32),
        pltpu.VMEM((PADC,), jnp.float32),
        pltpu.VMEM((PADC,), jnp.float32),
        pltpu.VMEM((L,), jnp.float32),
        pltpu.VMEM((NTILES, L), jnp.float32),
        pltpu.VMEM_SHARED((NTILES, L), jnp.float32),
        pltpu.SemaphoreType.DMA,
    ],
    compiler_params=pltpu.CompilerParams(needs_layout_passes=False),
)
def _sc_loss(u_hbm, ptr_hbm, xf_hbm, s_hbm, lz_hbm, out_hbm,
             u_v, ptr_v, lbl_v, s_v, lz_v, acc_v, sums_v, shared, sem):
    cid = lax.axis_index("c")
    sid = lax.axis_index("s")

    @pl.when(cid == 0)
    def _core0():
        pltpu.sync_copy(u_hbm.at[sid], u_v)
        pltpu.sync_copy(ptr_hbm, ptr_v.at[pl.ds(0, NSEG + 1)])
        lo = plsc.load_gather(ptr_v, [jnp.full((L,), sid, jnp.int32)])
        hi = plsc.load_gather(ptr_v, [jnp.full((L,), sid + 1, jnp.int32)])
        szf = (hi - lo).astype(jnp.float32)

        copies = []
        for j in range(NVEC):
            elem = ((u_v[j] * szf).astype(jnp.int32) + lo) * EMBED_DIM
            copies.append(
                pltpu.async_copy(xf_hbm.at[elem], lbl_v.at[pl.ds(j * L, L)], sem))
        pltpu.sync_copy(s_hbm.at[0], s_v)
        for cp in copies:
            cp.wait()

        iota = lax.iota(jnp.int32, L)
        acc = jnp.zeros((L,), jnp.float32)
        for j in range(NVEC):
            li = lbl_v[pl.ds(j * L, L)].astype(jnp.int32)
            sv = plsc.load_gather(s_v, [li])
            acc = acc + jnp.where((j * L) + iota < PER_SEG, sv, 0.0)
        acc_v[...] = acc
        pltpu.sync_copy(acc_v, shared.at[sid])
        plsc.subcore_barrier()

        @pl.when(sid == 0)
        def _reduce():
            pltpu.sync_copy(shared, sums_v)
            pltpu.sync_copy(lz_hbm.at[0], lz_v)
            tot = jnp.zeros((L,), jnp.float32)
            for r in range(NTILES):
                tot = tot + sums_v[r]
            total = jnp.sum(tot)
            lzv = lz_v[pl.ds(0, L)]
            acc_v[...] = lzv - total * (1.0 / TOTAL)
            pltpu.sync_copy(acc_v, out_hbm)


def kernel(x, ptr, W_model, b_model, W1, b1, W2, b2):
    s_arr, lz_arr = _head(b_model, W1, b1, W2, b2)

    u_c = jnp.asarray(_U_PAD)
    out = _sc_loss(u_c, ptr, x.reshape(-1), s_arr, lz_arr)
    return out[0]


# speedup vs baseline: 1.2705x; 1.0849x over previous
import functools

import numpy as np
import jax
import jax.numpy as jnp
from jax import lax
from jax.experimental import pallas as pl
from jax.experimental.pallas import tpu as pltpu
from jax.experimental.pallas import tpu_sc as plsc

EMBED_DIM = 128
HIDDEN = 256
NUM_CLASSES = 119
PADC = 128
SEG = 1024
MASK_RATE = 0.3
NSEG = 16
N = NSEG * SEG
PER_SEG = max(int(SEG * MASK_RATE), 1)
TOTAL = NSEG * PER_SEG

TASK_COL = 0
NTILES = 16
L = 16
NVEC = 20


def _np_uniform_threefry(seed: int, n: int) -> np.ndarray:
    def rotl(x, d):
        return ((x << np.uint32(d)) | (x >> np.uint32(32 - d))).astype(np.uint32)

    counts = np.arange(n, dtype=np.uint64)
    x = [(counts >> np.uint64(32)).astype(np.uint32),
         (counts & np.uint64(0xFFFFFFFF)).astype(np.uint32)]
    k0, k1 = np.uint32(0), np.uint32(seed)
    rotations = ((13, 15, 26, 6), (17, 29, 16, 24))
    ks = (k0, k1, np.uint32(k0 ^ k1 ^ 0x1BD11BDA))
    x[0] = (x[0] + ks[0]).astype(np.uint32)
    x[1] = (x[1] + ks[1]).astype(np.uint32)
    for i in range(5):
        for r in rotations[i % 2]:
            x[0] = (x[0] + x[1]).astype(np.uint32)
            x[1] = rotl(x[1], r) ^ x[0]
        x[0] = (x[0] + ks[(i + 1) % 3]).astype(np.uint32)
        x[1] = (x[1] + ks[(i + 2) % 3] + np.uint32(i + 1)).astype(np.uint32)
    bits = x[0] ^ x[1]
    f = ((bits >> np.uint32(9)) | np.uint32(0x3F800000)).view(np.float32) - np.float32(1.0)
    return np.maximum(np.float32(0.0), f)


_U = _np_uniform_threefry(42, TOTAL)
_U_PAD = np.zeros((NTILES, NVEC * L), dtype=np.float32)
_U_PAD[:, :PER_SEG] = _U.reshape(NTILES, PER_SEG)
_U_PAD = _U_PAD.reshape(NTILES, NVEC, L)


def _head_body(bm_ref, w1_ref, b1_ref, w2_ref, b2_ref, s_ref, lz_ref):
    rb = jnp.maximum(bm_ref[...].reshape(1, EMBED_DIM), 0.0)
    hid = jnp.dot(rb, w1_ref[...], preferred_element_type=jnp.float32)
    hid = jnp.maximum(hid + b1_ref[...].reshape(1, HIDDEN), 0.0)
    s = jnp.dot(hid, w2_ref[...], preferred_element_type=jnp.float32)
    s = s + b2_ref[...].reshape(1, NUM_CLASSES)
    m = jnp.max(s)
    lz = m + jnp.log(jnp.sum(jnp.exp(s - m)))
    sp = jnp.concatenate(
        [s, jnp.full((1, PADC - NUM_CLASSES), -1e30, jnp.float32)], axis=1)
    s_ref[...] = jnp.broadcast_to(sp, (8, PADC))
    lz_ref[...] = jnp.full((8, PADC), lz, dtype=jnp.float32)


_head = pl.pallas_call(
    _head_body,
    out_shape=[
        jax.ShapeDtypeStruct((8, PADC), jnp.float32),
        jax.ShapeDtypeStruct((8, PADC), jnp.float32),
    ],
)


_mesh = plsc.VectorSubcoreMesh(core_axis_name="c", subcore_axis_name="s",
                               num_cores=1)


@functools.partial(
    pl.kernel,
    mesh=_mesh,
    out_type=jax.ShapeDtypeStruct((L,), jnp.float32),
    scratch_types=[
        pltpu.VMEM((NVEC, L), jnp.float32),
        pltpu.VMEM((24,), jnp.int32),
        pltpu.VMEM((NVEC * L,), jnp.float32),
        pltpu.VMEM((PADC,), jnp.float32),
        pltpu.VMEM((PADC,), jnp.float32),
        pltpu.VMEM((L,), jnp.float32),
        pltpu.VMEM((NTILES, L), jnp.float32),
        pltpu.VMEM_SHARED((NTILES, L), jnp.float32),
        pltpu.SemaphoreType.DMA,
        pltpu.SemaphoreType.DMA,
        pltpu.SemaphoreType.DMA,
        pltpu.SemaphoreType.DMA,
    ],
    compiler_params=pltpu.CompilerParams(needs_layout_passes=False),
)
def _sc_loss(u_hbm, ptr_hbm, xf_hbm, s_hbm, lz_hbm, out_hbm,
             u_v, ptr_v, lbl_v, s_v, lz_v, acc_v, sums_v, shared,
             sem, sem_u, sem_p, sem_s):
    sid = lax.axis_index("s")

    u_cp = pltpu.async_copy(u_hbm.at[sid], u_v, sem_u)
    p_cp = pltpu.async_copy(ptr_hbm, ptr_v.at[pl.ds(0, NSEG + 1)], sem_p)
    s_cp = pltpu.async_copy(s_hbm.at[0], s_v, sem_s)
    u_cp.wait()
    p_cp.wait()
    lo = plsc.load_gather(ptr_v, [jnp.full((L,), sid, jnp.int32)])
    hi = plsc.load_gather(ptr_v, [jnp.full((L,), sid + 1, jnp.int32)])
    szf = (hi - lo).astype(jnp.float32)

    copies = []
    for j in range(NVEC):
        elem = ((u_v[j] * szf).astype(jnp.int32) + lo) * EMBED_DIM
        copies.append(
            pltpu.async_copy(xf_hbm.at[elem], lbl_v.at[pl.ds(j * L, L)], sem))
    s_cp.wait()
    for cp in copies:
        cp.wait()

    iota = lax.iota(jnp.int32, L)
    acc = jnp.zeros((L,), jnp.float32)
    for j in range(NVEC):
        li = lbl_v[pl.ds(j * L, L)].astype(jnp.int32)
        sv = plsc.load_gather(s_v, [li])
        acc = acc + jnp.where((j * L) + iota < PER_SEG, sv, 0.0)
    acc_v[...] = acc
    pltpu.sync_copy(acc_v, shared.at[sid])
    plsc.subcore_barrier()

    @pl.when(sid == 0)
    def _reduce():
        pltpu.sync_copy(shared, sums_v)
        pltpu.sync_copy(lz_hbm.at[0], lz_v)
        tot = jnp.zeros((L,), jnp.float32)
        for r in range(NTILES):
            tot = tot + sums_v[r]
        total = jnp.sum(tot)
        lzv = lz_v[pl.ds(0, L)]
        acc_v[...] = lzv - total * (1.0 / TOTAL)
        pltpu.sync_copy(acc_v, out_hbm)


def kernel(x, ptr, W_model, b_model, W1, b1, W2, b2):
    s_arr, lz_arr = _head(b_model, W1, b1, W2, b2)

    u_c = jnp.asarray(_U_PAD)
    out = _sc_loss(u_c, ptr, x.reshape(-1), s_arr, lz_arr)
    return out[0]
